# baseline (speedup 1.0000x reference)
import functools

import jax
import jax.numpy as jnp
from jax import lax
from jax.experimental import pallas as pl
from jax.experimental.pallas import tpu as pltpu
from jax.experimental.pallas import tpu_sc as plsc

N = 32768
NS = 16
L = 16
CHUNK = N // NS
NV = CHUNK // L

_mesh = plsc.VectorSubcoreMesh(
    core_axis_name="c", subcore_axis_name="s", num_cores=1
)


def _sc_cumsum_body(x_hbm, out_hbm, x_v, tot_v, all_v, shared):
    sid = lax.axis_index("s")
    base = sid * CHUNK

    pltpu.sync_copy(x_hbm.at[pl.ds(base, CHUNK)], x_v)

    def acc_body(j, acc):
        return acc + x_v[pl.ds(j * L, L)]

    acc = lax.fori_loop(0, NV, acc_body, jnp.zeros((L,), jnp.float32))
    total = jnp.sum(acc)

    tot_v[...] = jnp.zeros((L,), jnp.float32) + total
    pltpu.sync_copy(tot_v, shared.at[pl.ds(sid * L, L)])
    plsc.subcore_barrier()
    pltpu.sync_copy(shared, all_v)

    def off_body(k, off):
        row = all_v[pl.ds(k * L, L)]
        return off + jnp.where(k < sid, row, jnp.zeros((L,), jnp.float32))

    off = lax.fori_loop(0, NS, off_body, jnp.zeros((L,), jnp.float32))

    def scan_body(j, carry):
        v = x_v[pl.ds(j * L, L)]
        s = plsc.cumsum(v) + carry
        x_v[pl.ds(j * L, L)] = s
        return s[L - 1]

    lax.fori_loop(0, NV, scan_body, off[0])

    pltpu.sync_copy(x_v, out_hbm.at[pl.ds(base, CHUNK)])


_sc_cumsum = pl.kernel(
    _sc_cumsum_body,
    out_type=jax.ShapeDtypeStruct((N,), jnp.float32),
    mesh=_mesh,
    compiler_params=pltpu.CompilerParams(needs_layout_passes=False),
    scratch_types=[
        pltpu.VMEM((CHUNK,), jnp.float32),
        pltpu.VMEM((L,), jnp.float32),
        pltpu.VMEM((NS * L,), jnp.float32),
        pltpu.VMEM_SHARED((NS * L,), jnp.float32),
    ],
)


def kernel(mask_i):
    return _sc_cumsum(mask_i)
